# baseline (speedup 1.0000x reference)
import functools

import jax
import jax.numpy as jnp
from jax import lax
from jax.experimental import pallas as pl
from jax.experimental.pallas import tpu as pltpu
from jax.experimental.pallas import tpu_sc as plsc

DIM = 64
BATCH = 16384
TOT = 2 * BATCH
NC, NS, L = 2, 16, 16
NW = NC * NS
PER_W = TOT // NW
CHUNK = 128
N_CHUNKS = PER_W // CHUNK
GROUPS = CHUNK // L


def _transe_body(ih_hbm, ir_hbm, it_hbm, ent_hbm, rel_hbm, out_hbm,
                 ihv, irv, itv, hb, rb, tb, part, outb, sem):
    wid = lax.axis_index("s") * NC + lax.axis_index("c")
    base = wid * PER_W

    @pl.loop(0, N_CHUNKS)
    def _chunk(c):
        start = base + c * CHUNK
        pltpu.sync_copy(ih_hbm.at[pl.ds(start, CHUNK)], ihv)
        pltpu.sync_copy(ir_hbm.at[pl.ds(start, CHUNK)], irv)
        pltpu.sync_copy(it_hbm.at[pl.ds(start, CHUNK)], itv)
        ch = pltpu.async_copy(ent_hbm.at[ihv], hb, sem)
        cr = pltpu.async_copy(rel_hbm.at[irv], rb, sem)
        ct = pltpu.async_copy(ent_hbm.at[itv], tb, sem)
        ch.wait()
        cr.wait()
        ct.wait()

        @pl.loop(0, GROUPS)
        def _group(g):
            row0 = g * L
            for t in range(L):
                row = row0 + t
                acc = None
                for d in range(DIM // L):
                    sl = pl.ds(d * L, L)
                    v = jnp.abs(hb[row, sl] + rb[row, sl] - tb[row, sl])
                    acc = v if acc is None else acc + v
                part[pl.ds(t * L, L)] = acc
            rows = lax.iota(jnp.int32, L) * L
            red = plsc.load_gather(part, [rows])
            for j in range(1, L):
                red = red + plsc.load_gather(part, [rows + j])
            outb[pl.ds(row0, L)] = red

        pltpu.sync_copy(outb, out_hbm.at[pl.ds(start, CHUNK)])


@jax.jit
def kernel(positive_triplets, negative_triplets, entities_emb, relations_emb):
    trip = jnp.concatenate(
        [positive_triplets, negative_triplets], axis=0).astype(jnp.int32)
    ih = trip[:, 0]
    ir = trip[:, 1]
    it = trip[:, 2]

    mesh = plsc.VectorSubcoreMesh(
        core_axis_name="c", subcore_axis_name="s",
        num_cores=NC, num_subcores=NS)
    run = pl.kernel(
        _transe_body,
        out_type=jax.ShapeDtypeStruct((TOT,), jnp.float32),
        mesh=mesh,
        compiler_params=pltpu.CompilerParams(
            needs_layout_passes=False, use_tc_tiling_on_sc=False),
        scratch_types=[
            pltpu.VMEM((CHUNK,), jnp.int32),
            pltpu.VMEM((CHUNK,), jnp.int32),
            pltpu.VMEM((CHUNK,), jnp.int32),
            pltpu.VMEM((CHUNK, DIM), jnp.float32),
            pltpu.VMEM((CHUNK, DIM), jnp.float32),
            pltpu.VMEM((CHUNK, DIM), jnp.float32),
            pltpu.VMEM((L * L,), jnp.float32),
            pltpu.VMEM((CHUNK,), jnp.float32),
            pltpu.SemaphoreType.DMA,
        ],
    )
    out = run(ih, ir, it, entities_emb, relations_emb)
    return out[:BATCH], out[BATCH:]
